# baseline (speedup 1.0000x reference)
import functools

import jax
import jax.numpy as jnp
from jax import lax
from jax.experimental import pallas as pl
from jax.experimental.pallas import tpu as pltpu
from jax.experimental.pallas import tpu_sc as plsc

N = 10000
D = 128
E = 80000

NC = 2
NS = 16
CHUNK = 128
N_PAD = 10240
E_PAD = 81920
NCHUNK = E_PAD // CHUNK
ROWS_PER_TILE = N_PAD // NS

_mesh = plsc.VectorSubcoreMesh(core_axis_name="c", subcore_axis_name="s")


_CPT_DEG = NCHUNK // NS


def _deg_one_conv(s, dref, out_ref, zcol_h, idx_v, ones_v, sh):
    pltpu.sync_copy(zcol_h, sh.at[pl.ds(s * ROWS_PER_TILE, ROWS_PER_TILE)])
    plsc.subcore_barrier()

    pltpu.sync_copy(dref.at[pl.ds(s * _CPT_DEG, _CPT_DEG)], idx_v)

    @pl.loop(0, _CPT_DEG)
    def _(j):
        pltpu.sync_copy(ones_v, sh.at[idx_v.at[j]], add=True)

    plsc.subcore_barrier()
    pltpu.sync_copy(
        sh.at[pl.ds(s * ROWS_PER_TILE, ROWS_PER_TILE)],
        out_ref.at[pl.ds(s * ROWS_PER_TILE, ROWS_PER_TILE)],
    )
    plsc.subcore_barrier()


@functools.partial(
    pl.kernel,
    out_type=tuple(jax.ShapeDtypeStruct((N_PAD,), jnp.float32) for _ in range(4)),
    mesh=_mesh,
    scratch_types=[
        pltpu.VMEM((_CPT_DEG, CHUNK), jnp.int32),
        pltpu.VMEM((CHUNK,), jnp.float32),
        pltpu.VMEM_SHARED((N_PAD,), jnp.float32),
    ],
)
def _deg_kernel(d0, d1, d2, d3, ones_h, zcol_h,
                out0, out1, out2, out3,
                idx_v, ones_v, sh):
    c = lax.axis_index("c")
    s = lax.axis_index("s")

    pltpu.sync_copy(ones_h, ones_v)

    @pl.when(c == 0)
    def _():
        _deg_one_conv(s, d0, out0, zcol_h, idx_v, ones_v, sh)
        _deg_one_conv(s, d1, out1, zcol_h, idx_v, ones_v, sh)

    @pl.when(c == 1)
    def _():
        _deg_one_conv(s, d2, out2, zcol_h, idx_v, ones_v, sh)
        _deg_one_conv(s, d3, out3, zcol_h, idx_v, ones_v, sh)


_CPT_AGG = NCHUNK // NS


_NB = 2


def _agg_one_conv(s, g_ref, src_ref, dst_ref, zrows_h, out_ref,
                  sidx_v, didx_v, rows_v, acc_sh, gsems, ssems):
    pltpu.sync_copy(zrows_h, acc_sh.at[pl.ds(s * ROWS_PER_TILE, ROWS_PER_TILE)])
    plsc.subcore_barrier()

    pltpu.sync_copy(src_ref.at[pl.ds(s * _CPT_AGG, _CPT_AGG)], sidx_v)
    pltpu.sync_copy(dst_ref.at[pl.ds(s * _CPT_AGG, _CPT_AGG)], didx_v)

    for b in range(_NB):
        pltpu.async_copy(g_ref.at[sidx_v.at[b]], rows_v.at[b], gsems[b])

    @pl.loop(0, _CPT_AGG, step=_NB)
    def _(j):
        for b in range(_NB):
            jj = j + b
            pltpu.make_async_copy(
                g_ref.at[sidx_v.at[jj]], rows_v.at[b], gsems[b]).wait()
            pltpu.async_copy(
                rows_v.at[b], acc_sh.at[didx_v.at[jj]], ssems[b], add=True)
        for b in range(_NB):
            jj = j + b
            pltpu.make_async_copy(
                rows_v.at[b], acc_sh.at[didx_v.at[jj]], ssems[b]).wait()

            @pl.when(jj + _NB < _CPT_AGG)
            def _():
                pltpu.async_copy(
                    g_ref.at[sidx_v.at[jj + _NB]], rows_v.at[b], gsems[b])

    plsc.subcore_barrier()
    pltpu.sync_copy(
        acc_sh.at[pl.ds(s * ROWS_PER_TILE, ROWS_PER_TILE)],
        out_ref.at[pl.ds(s * ROWS_PER_TILE, ROWS_PER_TILE)],
    )
    plsc.subcore_barrier()


@functools.partial(
    pl.kernel,
    out_type=tuple(jax.ShapeDtypeStruct((N_PAD, D), jnp.float32) for _ in range(4)),
    mesh=_mesh,
    scratch_types=[
        pltpu.VMEM((_CPT_AGG, CHUNK), jnp.int32),
        pltpu.VMEM((_CPT_AGG, CHUNK), jnp.int32),
        pltpu.VMEM((_NB, CHUNK, D), jnp.float32),
        pltpu.VMEM_SHARED((N_PAD, D), jnp.float32),
    ] + [pltpu.SemaphoreType.DMA] * (2 * _NB),
)
def _agg_kernel(g0, g1, g2, g3, s0, s1, s2, s3, t0, t1, t2, t3, zrows_h,
                o0, o1, o2, o3,
                sidx_v, didx_v, rows_v, acc_sh, *sems):
    c = lax.axis_index("c")
    s = lax.axis_index("s")
    gsems, ssems = sems[:_NB], sems[_NB:]

    @pl.when(c == 0)
    def _():
        _agg_one_conv(s, g0, s0, t0, zrows_h, o0, sidx_v, didx_v, rows_v, acc_sh, gsems, ssems)
        _agg_one_conv(s, g1, s1, t1, zrows_h, o1, sidx_v, didx_v, rows_v, acc_sh, gsems, ssems)

    @pl.when(c == 1)
    def _():
        _agg_one_conv(s, g2, s2, t2, zrows_h, o2, sidx_v, didx_v, rows_v, acc_sh, gsems, ssems)
        _agg_one_conv(s, g3, s3, t3, zrows_h, o3, sidx_v, didx_v, rows_v, acc_sh, gsems, ssems)


BLK = 512


def _scale_body(x_ref, w1_ref, w2_ref, degp_ref, g0, g1, g2, g3):
    h1 = jnp.dot(x_ref[...], w1_ref[...], preferred_element_type=jnp.float32)
    h2 = jnp.dot(x_ref[...], w2_ref[...], preferred_element_type=jnp.float32)
    dinv = lax.rsqrt(degp_ref[...] + 1.0)
    g0[...] = h1 * dinv[0][:, None]
    g1[...] = h1 * dinv[1][:, None]
    g2[...] = h2 * dinv[2][:, None]
    g3[...] = h2 * dinv[3][:, None]


def _scale_call(x_pad, W1, W2, degp):
    grid = (N_PAD // BLK,)
    gspec = pl.BlockSpec((BLK, D), lambda i: (i, 0))
    return pl.pallas_call(
        _scale_body,
        grid=grid,
        in_specs=[
            pl.BlockSpec((BLK, D), lambda i: (i, 0)),
            pl.BlockSpec((D, D), lambda i: (0, 0)),
            pl.BlockSpec((D, D), lambda i: (0, 0)),
            pl.BlockSpec((8, BLK), lambda i: (0, i)),
        ],
        out_specs=[gspec, gspec, gspec, gspec],
        out_shape=[jax.ShapeDtypeStruct((N_PAD, D), jnp.float32)] * 4,
    )(x_pad, W1, W2, degp)


def _finish_body(a0, a1, a2, a3, g0, g1, g2, g3, degp_ref,
                 b1_ref, b2_ref, wt_top_ref, wt_bot_ref, bt_ref,
                 xa_ref, xb_ref):
    dinv = lax.rsqrt(degp_ref[...] + 1.0)
    relu = lambda v: jnp.maximum(v, 0.0)
    o0 = relu(dinv[0][:, None] * (a0[...] + g0[...]) + b1_ref[...])
    o1 = relu(dinv[1][:, None] * (a1[...] + g1[...]) + b1_ref[...])
    o2 = relu(dinv[2][:, None] * (a2[...] + g2[...]) + b2_ref[...])
    o3 = relu(dinv[3][:, None] * (a3[...] + g3[...]) + b2_ref[...])
    wt_top = wt_top_ref[...]
    wt_bot = wt_bot_ref[...]
    xa_ref[...] = relu(
        jnp.dot(o0, wt_top, preferred_element_type=jnp.float32)
        + jnp.dot(o2, wt_bot, preferred_element_type=jnp.float32)
        + bt_ref[...])
    xb_ref[...] = relu(
        jnp.dot(o1, wt_top, preferred_element_type=jnp.float32)
        + jnp.dot(o3, wt_bot, preferred_element_type=jnp.float32)
        + bt_ref[...])


def _finish_call(accs, gs, degp, b1, b2, Wt, bt):
    grid = (N_PAD // BLK,)
    nspec = pl.BlockSpec((BLK, D), lambda i: (i, 0))
    wspec = pl.BlockSpec((D, D), lambda i: (0, 0))
    bspec = pl.BlockSpec((1, D), lambda i: (0, 0))
    return pl.pallas_call(
        _finish_body,
        grid=grid,
        in_specs=[nspec] * 8 + [
            pl.BlockSpec((8, BLK), lambda i: (0, i)),
            bspec, bspec, wspec, wspec, bspec,
        ],
        out_specs=[nspec, nspec],
        out_shape=[jax.ShapeDtypeStruct((N_PAD, D), jnp.float32)] * 2,
    )(*accs, *gs, degp, b1.reshape(1, D), b2.reshape(1, D),
      Wt[:D], Wt[D:], bt.reshape(1, D))


def _pad_edges(ei):
    src = ei[0].astype(jnp.int32)
    dst = ei[1].astype(jnp.int32)
    pad = jnp.full((E_PAD - E,), N, jnp.int32)
    src = jnp.concatenate([src, pad]).reshape(NCHUNK, CHUNK)
    dst = jnp.concatenate([dst, pad]).reshape(NCHUNK, CHUNK)
    return src, dst


def kernel(x, view_a_pos, view_a_neg, view_b_pos, view_b_neg,
           W1, b1, W2, b2, Wt, bt):
    edges = [_pad_edges(v) for v in
             (view_a_pos, view_b_pos, view_a_neg, view_b_neg)]
    srcs = [e[0] for e in edges]
    dsts = [e[1] for e in edges]

    x_pad = jnp.pad(x, ((0, N_PAD - N), (0, 0)))
    ones_h = jnp.ones((CHUNK,), jnp.float32)
    zcol_h = jnp.zeros((ROWS_PER_TILE,), jnp.float32)
    zrows_h = jnp.zeros((ROWS_PER_TILE, D), jnp.float32)

    deg4 = _deg_kernel(*dsts, ones_h, zcol_h)
    degp = jnp.concatenate(
        [jnp.stack(deg4), jnp.zeros((4, N_PAD), jnp.float32)], axis=0)
    gs = _scale_call(x_pad, W1, W2, degp)
    accs = _agg_kernel(*gs, *srcs, *dsts, zrows_h)
    xa, xb = _finish_call(accs, gs, degp, b1, b2, Wt, bt)
    return xa[:N], xb[:N]


# speedup vs baseline: 1.0480x; 1.0480x over previous
import functools

import jax
import jax.numpy as jnp
from jax import lax
from jax.experimental import pallas as pl
from jax.experimental.pallas import tpu as pltpu
from jax.experimental.pallas import tpu_sc as plsc

N = 10000
D = 128
E = 80000

NC = 2
NS = 16
CHUNK = 128
N_PAD = 10240
E_PAD = 81920
NCHUNK = E_PAD // CHUNK
ROWS_PER_TILE = N_PAD // NS

_mesh = plsc.VectorSubcoreMesh(core_axis_name="c", subcore_axis_name="s")


_CPT_DEG = NCHUNK // NS


def _deg_one_conv(s, dref, out_ref, zcol_h, idx_v, ones_v, sh):
    pltpu.sync_copy(zcol_h, sh.at[pl.ds(s * ROWS_PER_TILE, ROWS_PER_TILE)])
    plsc.subcore_barrier()

    pltpu.sync_copy(dref.at[pl.ds(s * _CPT_DEG, _CPT_DEG)], idx_v)

    @pl.loop(0, _CPT_DEG)
    def _(j):
        pltpu.sync_copy(ones_v, sh.at[idx_v.at[j]], add=True)

    plsc.subcore_barrier()
    pltpu.sync_copy(
        sh.at[pl.ds(s * ROWS_PER_TILE, ROWS_PER_TILE)],
        out_ref.at[pl.ds(s * ROWS_PER_TILE, ROWS_PER_TILE)],
    )
    plsc.subcore_barrier()


@functools.partial(
    pl.kernel,
    out_type=tuple(jax.ShapeDtypeStruct((N_PAD,), jnp.float32) for _ in range(4)),
    mesh=_mesh,
    scratch_types=[
        pltpu.VMEM((_CPT_DEG, CHUNK), jnp.int32),
        pltpu.VMEM((CHUNK,), jnp.float32),
        pltpu.VMEM_SHARED((N_PAD,), jnp.float32),
    ],
)
def _deg_kernel(d0, d1, d2, d3, ones_h, zcol_h,
                out0, out1, out2, out3,
                idx_v, ones_v, sh):
    c = lax.axis_index("c")
    s = lax.axis_index("s")

    pltpu.sync_copy(ones_h, ones_v)

    @pl.when(c == 0)
    def _():
        _deg_one_conv(s, d0, out0, zcol_h, idx_v, ones_v, sh)
        _deg_one_conv(s, d1, out1, zcol_h, idx_v, ones_v, sh)

    @pl.when(c == 1)
    def _():
        _deg_one_conv(s, d2, out2, zcol_h, idx_v, ones_v, sh)
        _deg_one_conv(s, d3, out3, zcol_h, idx_v, ones_v, sh)


_CPT_AGG = NCHUNK // NS


_NB = 2


def _agg_one_conv(s, g_ref, src_ref, dst_ref, zrows_h, out_ref,
                  sidx_v, didx_v, rows_v, acc_sh, gsems, ssems):
    pltpu.sync_copy(zrows_h, acc_sh.at[pl.ds(s * ROWS_PER_TILE, ROWS_PER_TILE)])
    plsc.subcore_barrier()

    pltpu.sync_copy(src_ref.at[pl.ds(s * _CPT_AGG, _CPT_AGG)], sidx_v)
    pltpu.sync_copy(dst_ref.at[pl.ds(s * _CPT_AGG, _CPT_AGG)], didx_v)

    for b in range(_NB):
        pltpu.async_copy(g_ref.at[sidx_v.at[b]], rows_v.at[b], gsems[b])

    @pl.loop(0, _CPT_AGG, step=_NB)
    def _(j):
        for b in range(_NB):
            jj = j + b
            pltpu.make_async_copy(
                g_ref.at[sidx_v.at[jj]], rows_v.at[b], gsems[b]).wait()
            pltpu.sync_copy(rows_v.at[b], acc_sh.at[didx_v.at[jj]], add=True)

            @pl.when(jj + _NB < _CPT_AGG)
            def _():
                pltpu.async_copy(
                    g_ref.at[sidx_v.at[jj + _NB]], rows_v.at[b], gsems[b])

    plsc.subcore_barrier()
    pltpu.sync_copy(
        acc_sh.at[pl.ds(s * ROWS_PER_TILE, ROWS_PER_TILE)],
        out_ref.at[pl.ds(s * ROWS_PER_TILE, ROWS_PER_TILE)],
    )
    plsc.subcore_barrier()


@functools.partial(
    pl.kernel,
    out_type=tuple(jax.ShapeDtypeStruct((N_PAD, D), jnp.float32) for _ in range(4)),
    mesh=_mesh,
    scratch_types=[
        pltpu.VMEM((_CPT_AGG, CHUNK), jnp.int32),
        pltpu.VMEM((_CPT_AGG, CHUNK), jnp.int32),
        pltpu.VMEM((_NB, CHUNK, D), jnp.float32),
        pltpu.VMEM_SHARED((N_PAD, D), jnp.float32),
    ] + [pltpu.SemaphoreType.DMA] * (2 * _NB),
)
def _agg_kernel(g0, g1, g2, g3, s0, s1, s2, s3, t0, t1, t2, t3, zrows_h,
                o0, o1, o2, o3,
                sidx_v, didx_v, rows_v, acc_sh, *sems):
    c = lax.axis_index("c")
    s = lax.axis_index("s")
    gsems, ssems = sems[:_NB], sems[_NB:]

    @pl.when(c == 0)
    def _():
        _agg_one_conv(s, g0, s0, t0, zrows_h, o0, sidx_v, didx_v, rows_v, acc_sh, gsems, ssems)
        _agg_one_conv(s, g1, s1, t1, zrows_h, o1, sidx_v, didx_v, rows_v, acc_sh, gsems, ssems)

    @pl.when(c == 1)
    def _():
        _agg_one_conv(s, g2, s2, t2, zrows_h, o2, sidx_v, didx_v, rows_v, acc_sh, gsems, ssems)
        _agg_one_conv(s, g3, s3, t3, zrows_h, o3, sidx_v, didx_v, rows_v, acc_sh, gsems, ssems)


BLK = 512


def _scale_body(x_ref, w1_ref, w2_ref, degp_ref, g0, g1, g2, g3):
    h1 = jnp.dot(x_ref[...], w1_ref[...], preferred_element_type=jnp.float32)
    h2 = jnp.dot(x_ref[...], w2_ref[...], preferred_element_type=jnp.float32)
    dinv = lax.rsqrt(degp_ref[...] + 1.0)
    g0[...] = h1 * dinv[0][:, None]
    g1[...] = h1 * dinv[1][:, None]
    g2[...] = h2 * dinv[2][:, None]
    g3[...] = h2 * dinv[3][:, None]


def _scale_call(x_pad, W1, W2, degp):
    grid = (N_PAD // BLK,)
    gspec = pl.BlockSpec((BLK, D), lambda i: (i, 0))
    return pl.pallas_call(
        _scale_body,
        grid=grid,
        in_specs=[
            pl.BlockSpec((BLK, D), lambda i: (i, 0)),
            pl.BlockSpec((D, D), lambda i: (0, 0)),
            pl.BlockSpec((D, D), lambda i: (0, 0)),
            pl.BlockSpec((8, BLK), lambda i: (0, i)),
        ],
        out_specs=[gspec, gspec, gspec, gspec],
        out_shape=[jax.ShapeDtypeStruct((N_PAD, D), jnp.float32)] * 4,
    )(x_pad, W1, W2, degp)


def _finish_body(a0, a1, a2, a3, g0, g1, g2, g3, degp_ref,
                 b1_ref, b2_ref, wt_top_ref, wt_bot_ref, bt_ref,
                 xa_ref, xb_ref):
    dinv = lax.rsqrt(degp_ref[...] + 1.0)
    relu = lambda v: jnp.maximum(v, 0.0)
    o0 = relu(dinv[0][:, None] * (a0[...] + g0[...]) + b1_ref[...])
    o1 = relu(dinv[1][:, None] * (a1[...] + g1[...]) + b1_ref[...])
    o2 = relu(dinv[2][:, None] * (a2[...] + g2[...]) + b2_ref[...])
    o3 = relu(dinv[3][:, None] * (a3[...] + g3[...]) + b2_ref[...])
    wt_top = wt_top_ref[...]
    wt_bot = wt_bot_ref[...]
    xa_ref[...] = relu(
        jnp.dot(o0, wt_top, preferred_element_type=jnp.float32)
        + jnp.dot(o2, wt_bot, preferred_element_type=jnp.float32)
        + bt_ref[...])
    xb_ref[...] = relu(
        jnp.dot(o1, wt_top, preferred_element_type=jnp.float32)
        + jnp.dot(o3, wt_bot, preferred_element_type=jnp.float32)
        + bt_ref[...])


def _finish_call(accs, gs, degp, b1, b2, Wt, bt):
    grid = (N_PAD // BLK,)
    nspec = pl.BlockSpec((BLK, D), lambda i: (i, 0))
    wspec = pl.BlockSpec((D, D), lambda i: (0, 0))
    bspec = pl.BlockSpec((1, D), lambda i: (0, 0))
    return pl.pallas_call(
        _finish_body,
        grid=grid,
        in_specs=[nspec] * 8 + [
            pl.BlockSpec((8, BLK), lambda i: (0, i)),
            bspec, bspec, wspec, wspec, bspec,
        ],
        out_specs=[nspec, nspec],
        out_shape=[jax.ShapeDtypeStruct((N, D), jnp.float32)] * 2,
    )(*accs, *gs, degp, b1.reshape(1, D), b2.reshape(1, D),
      Wt[:D], Wt[D:], bt.reshape(1, D))


def _pad_edges(ei):
    src = ei[0].astype(jnp.int32)
    dst = ei[1].astype(jnp.int32)
    pad = jnp.full((E_PAD - E,), N, jnp.int32)
    src = jnp.concatenate([src, pad]).reshape(NCHUNK, CHUNK)
    dst = jnp.concatenate([dst, pad]).reshape(NCHUNK, CHUNK)
    return src, dst


def kernel(x, view_a_pos, view_a_neg, view_b_pos, view_b_neg,
           W1, b1, W2, b2, Wt, bt):
    edges = [_pad_edges(v) for v in
             (view_a_pos, view_b_pos, view_a_neg, view_b_neg)]
    srcs = [e[0] for e in edges]
    dsts = [e[1] for e in edges]

    x_pad = jnp.pad(x, ((0, N_PAD - N), (0, 0)))
    ones_h = jnp.ones((CHUNK,), jnp.float32)
    zcol_h = jnp.zeros((ROWS_PER_TILE,), jnp.float32)
    zrows_h = jnp.zeros((ROWS_PER_TILE, D), jnp.float32)

    deg4 = _deg_kernel(*dsts, ones_h, zcol_h)
    degp = jnp.concatenate(
        [jnp.stack(deg4), jnp.zeros((4, N_PAD), jnp.float32)], axis=0)
    gs = _scale_call(x_pad, W1, W2, degp)
    accs = _agg_kernel(*gs, *srcs, *dsts, zrows_h)
    xa, xb = _finish_call(accs, gs, degp, b1, b2, Wt, bt)
    return xa, xb


# speedup vs baseline: 1.8035x; 1.7208x over previous
import functools

import jax
import jax.numpy as jnp
from jax import lax
from jax.experimental import pallas as pl
from jax.experimental.pallas import tpu as pltpu
from jax.experimental.pallas import tpu_sc as plsc

N = 10000
D = 128
E = 80000

NC = 2
NS = 16
CHUNK = 125
N_PAD = 10240
NCHUNK = E // CHUNK
ROWS_PER_TILE = N_PAD // NS

_mesh = plsc.VectorSubcoreMesh(core_axis_name="c", subcore_axis_name="s")


_CPT_DEG = NCHUNK // NS


def _deg_one_conv(s, dref, out_ref, zcol_h, idx_v, ones_v, sh):
    pltpu.sync_copy(zcol_h, sh.at[pl.ds(s * ROWS_PER_TILE, ROWS_PER_TILE)])
    plsc.subcore_barrier()

    pltpu.sync_copy(dref.at[pl.ds(s * _CPT_DEG, _CPT_DEG)], idx_v)

    @pl.loop(0, _CPT_DEG)
    def _(j):
        pltpu.sync_copy(ones_v, sh.at[idx_v.at[j]], add=True)

    plsc.subcore_barrier()
    pltpu.sync_copy(
        sh.at[pl.ds(s * ROWS_PER_TILE, ROWS_PER_TILE)],
        out_ref.at[pl.ds(s * ROWS_PER_TILE, ROWS_PER_TILE)],
    )
    plsc.subcore_barrier()


@functools.partial(
    pl.kernel,
    out_type=tuple(jax.ShapeDtypeStruct((N_PAD,), jnp.float32) for _ in range(4)),
    mesh=_mesh,
    scratch_types=[
        pltpu.VMEM((_CPT_DEG, CHUNK), jnp.int32),
        pltpu.VMEM((CHUNK,), jnp.float32),
        pltpu.VMEM_SHARED((N_PAD,), jnp.float32),
    ],
)
def _deg_kernel(d0, d1, d2, d3, ones_h, zcol_h,
                out0, out1, out2, out3,
                idx_v, ones_v, sh):
    c = lax.axis_index("c")
    s = lax.axis_index("s")

    pltpu.sync_copy(ones_h, ones_v)

    @pl.when(c == 0)
    def _():
        _deg_one_conv(s, d0, out0, zcol_h, idx_v, ones_v, sh)
        _deg_one_conv(s, d1, out1, zcol_h, idx_v, ones_v, sh)

    @pl.when(c == 1)
    def _():
        _deg_one_conv(s, d2, out2, zcol_h, idx_v, ones_v, sh)
        _deg_one_conv(s, d3, out3, zcol_h, idx_v, ones_v, sh)


_CPT_AGG = NCHUNK // NS


_NB = 2


def _agg_one_conv(s, g_ref, src_ref, dst_ref, zrows_h, out_ref,
                  sidx_v, didx_v, rows_v, acc_sh, gsems, ssems):
    pltpu.sync_copy(zrows_h, acc_sh.at[pl.ds(s * ROWS_PER_TILE, ROWS_PER_TILE)])
    plsc.subcore_barrier()

    pltpu.sync_copy(src_ref.at[pl.ds(s * _CPT_AGG, _CPT_AGG)], sidx_v)
    pltpu.sync_copy(dst_ref.at[pl.ds(s * _CPT_AGG, _CPT_AGG)], didx_v)

    for b in range(_NB):
        pltpu.async_copy(g_ref.at[sidx_v.at[b]], rows_v.at[b], gsems[b])

    @pl.loop(0, _CPT_AGG, step=_NB)
    def _(j):
        for b in range(_NB):
            jj = j + b
            pltpu.make_async_copy(
                g_ref.at[sidx_v.at[jj]], rows_v.at[b], gsems[b]).wait()
            pltpu.sync_copy(rows_v.at[b], acc_sh.at[didx_v.at[jj]], add=True)

            @pl.when(jj + _NB < _CPT_AGG)
            def _():
                pltpu.async_copy(
                    g_ref.at[sidx_v.at[jj + _NB]], rows_v.at[b], gsems[b])

    plsc.subcore_barrier()
    pltpu.sync_copy(
        acc_sh.at[pl.ds(s * ROWS_PER_TILE, ROWS_PER_TILE)],
        out_ref.at[pl.ds(s * ROWS_PER_TILE, ROWS_PER_TILE)],
    )
    plsc.subcore_barrier()


@functools.partial(
    pl.kernel,
    out_type=tuple(jax.ShapeDtypeStruct((N_PAD, D), jnp.float32) for _ in range(4)),
    mesh=_mesh,
    scratch_types=[
        pltpu.VMEM((_CPT_AGG, CHUNK), jnp.int32),
        pltpu.VMEM((_CPT_AGG, CHUNK), jnp.int32),
        pltpu.VMEM((_NB, CHUNK, D), jnp.float32),
        pltpu.VMEM_SHARED((N_PAD, D), jnp.float32),
    ] + [pltpu.SemaphoreType.DMA] * (2 * _NB),
)
def _agg_kernel(g0, g1, g2, g3, s0, s1, s2, s3, t0, t1, t2, t3, zrows_h,
                o0, o1, o2, o3,
                sidx_v, didx_v, rows_v, acc_sh, *sems):
    c = lax.axis_index("c")
    s = lax.axis_index("s")
    gsems, ssems = sems[:_NB], sems[_NB:]

    @pl.when(c == 0)
    def _():
        _agg_one_conv(s, g0, s0, t0, zrows_h, o0, sidx_v, didx_v, rows_v, acc_sh, gsems, ssems)
        _agg_one_conv(s, g1, s1, t1, zrows_h, o1, sidx_v, didx_v, rows_v, acc_sh, gsems, ssems)

    @pl.when(c == 1)
    def _():
        _agg_one_conv(s, g2, s2, t2, zrows_h, o2, sidx_v, didx_v, rows_v, acc_sh, gsems, ssems)
        _agg_one_conv(s, g3, s3, t3, zrows_h, o3, sidx_v, didx_v, rows_v, acc_sh, gsems, ssems)


BLK = 512


def _scale_body(x_ref, w1_ref, w2_ref, degp_ref, g0, g1, g2, g3):
    h1 = jnp.dot(x_ref[...], w1_ref[...], preferred_element_type=jnp.float32)
    h2 = jnp.dot(x_ref[...], w2_ref[...], preferred_element_type=jnp.float32)
    dinv = lax.rsqrt(degp_ref[...] + 1.0)
    g0[...] = h1 * dinv[0][:, None]
    g1[...] = h1 * dinv[1][:, None]
    g2[...] = h2 * dinv[2][:, None]
    g3[...] = h2 * dinv[3][:, None]


def _scale_call(x_pad, W1, W2, degp):
    grid = (N_PAD // BLK,)
    gspec = pl.BlockSpec((BLK, D), lambda i: (i, 0))
    return pl.pallas_call(
        _scale_body,
        grid=grid,
        in_specs=[
            pl.BlockSpec((BLK, D), lambda i: (i, 0)),
            pl.BlockSpec((D, D), lambda i: (0, 0)),
            pl.BlockSpec((D, D), lambda i: (0, 0)),
            pl.BlockSpec((8, BLK), lambda i: (0, i)),
        ],
        out_specs=[gspec, gspec, gspec, gspec],
        out_shape=[jax.ShapeDtypeStruct((N_PAD, D), jnp.float32)] * 4,
    )(x_pad, W1, W2, degp)


def _finish_body(a0, a1, a2, a3, g0, g1, g2, g3, degp_ref,
                 b1_ref, b2_ref, wt_top_ref, wt_bot_ref, bt_ref,
                 xa_ref, xb_ref):
    dinv = lax.rsqrt(degp_ref[...] + 1.0)
    relu = lambda v: jnp.maximum(v, 0.0)
    o0 = relu(dinv[0][:, None] * (a0[...] + g0[...]) + b1_ref[...])
    o1 = relu(dinv[1][:, None] * (a1[...] + g1[...]) + b1_ref[...])
    o2 = relu(dinv[2][:, None] * (a2[...] + g2[...]) + b2_ref[...])
    o3 = relu(dinv[3][:, None] * (a3[...] + g3[...]) + b2_ref[...])
    wt_top = wt_top_ref[...]
    wt_bot = wt_bot_ref[...]
    xa_ref[...] = relu(
        jnp.dot(o0, wt_top, preferred_element_type=jnp.float32)
        + jnp.dot(o2, wt_bot, preferred_element_type=jnp.float32)
        + bt_ref[...])
    xb_ref[...] = relu(
        jnp.dot(o1, wt_top, preferred_element_type=jnp.float32)
        + jnp.dot(o3, wt_bot, preferred_element_type=jnp.float32)
        + bt_ref[...])


def _finish_call(accs, gs, degp, b1, b2, Wt, bt):
    grid = (N_PAD // BLK,)
    nspec = pl.BlockSpec((BLK, D), lambda i: (i, 0))
    wspec = pl.BlockSpec((D, D), lambda i: (0, 0))
    bspec = pl.BlockSpec((1, D), lambda i: (0, 0))
    return pl.pallas_call(
        _finish_body,
        grid=grid,
        in_specs=[nspec] * 8 + [
            pl.BlockSpec((8, BLK), lambda i: (0, i)),
            bspec, bspec, wspec, wspec, bspec,
        ],
        out_specs=[nspec, nspec],
        out_shape=[jax.ShapeDtypeStruct((N_PAD, D), jnp.float32)] * 2,
    )(*accs, *gs, degp, b1.reshape(1, D), b2.reshape(1, D),
      Wt[:D], Wt[D:], bt.reshape(1, D))


def _pad_edges(ei):
    src = ei[0].astype(jnp.int32).reshape(NCHUNK, CHUNK)
    dst = ei[1].astype(jnp.int32).reshape(NCHUNK, CHUNK)
    return src, dst


def kernel(x, view_a_pos, view_a_neg, view_b_pos, view_b_neg,
           W1, b1, W2, b2, Wt, bt):
    edges = [_pad_edges(v) for v in
             (view_a_pos, view_b_pos, view_a_neg, view_b_neg)]
    srcs = [e[0] for e in edges]
    dsts = [e[1] for e in edges]

    x_pad = jnp.pad(x, ((0, N_PAD - N), (0, 0)))
    ones_h = jnp.ones((CHUNK,), jnp.float32)
    zcol_h = jnp.zeros((ROWS_PER_TILE,), jnp.float32)
    zrows_h = jnp.zeros((ROWS_PER_TILE, D), jnp.float32)

    deg4 = _deg_kernel(*dsts, ones_h, zcol_h)
    degp = jnp.concatenate(
        [jnp.stack(deg4), jnp.zeros((4, N_PAD), jnp.float32)], axis=0)
    gs = _scale_call(x_pad, W1, W2, degp)
    accs = _agg_kernel(*gs, *srcs, *dsts, zrows_h)
    xa, xb = _finish_call(accs, gs, degp, b1, b2, Wt, bt)
    return xa[:N], xb[:N]


# speedup vs baseline: 1.9136x; 1.0611x over previous
import functools

import jax
import jax.numpy as jnp
from jax import lax
from jax.experimental import pallas as pl
from jax.experimental.pallas import tpu as pltpu
from jax.experimental.pallas import tpu_sc as plsc

N = 10000
D = 128
E = 80000

NC = 2
NS = 16
CHUNK = 125
N_PAD = 10240
NCHUNK = E // CHUNK
ROWS_PER_TILE = N_PAD // NS

_mesh = plsc.VectorSubcoreMesh(core_axis_name="c", subcore_axis_name="s")


_CPT_DEG = NCHUNK // NS


def _deg_one_conv(s, dref, out_ref, zcol_h, idx_v, ones_v, sh):
    pltpu.sync_copy(zcol_h, sh.at[pl.ds(s * ROWS_PER_TILE, ROWS_PER_TILE)])
    plsc.subcore_barrier()

    pltpu.sync_copy(dref.at[pl.ds(s * _CPT_DEG, _CPT_DEG)], idx_v)

    @pl.loop(0, _CPT_DEG)
    def _(j):
        pltpu.sync_copy(ones_v, sh.at[idx_v.at[j]], add=True)

    plsc.subcore_barrier()
    pltpu.sync_copy(
        sh.at[pl.ds(s * ROWS_PER_TILE, ROWS_PER_TILE)],
        out_ref.at[pl.ds(s * ROWS_PER_TILE, ROWS_PER_TILE)],
    )
    plsc.subcore_barrier()


@functools.partial(
    pl.kernel,
    out_type=tuple(jax.ShapeDtypeStruct((N_PAD,), jnp.float32) for _ in range(4)),
    mesh=_mesh,
    scratch_types=[
        pltpu.VMEM((_CPT_DEG, CHUNK), jnp.int32),
        pltpu.VMEM((CHUNK,), jnp.float32),
        pltpu.VMEM_SHARED((N_PAD,), jnp.float32),
    ],
)
def _deg_kernel(d0, d1, d2, d3, ones_h, zcol_h,
                out0, out1, out2, out3,
                idx_v, ones_v, sh):
    c = lax.axis_index("c")
    s = lax.axis_index("s")

    pltpu.sync_copy(ones_h, ones_v)

    @pl.when(c == 0)
    def _():
        _deg_one_conv(s, d0, out0, zcol_h, idx_v, ones_v, sh)
        _deg_one_conv(s, d1, out1, zcol_h, idx_v, ones_v, sh)

    @pl.when(c == 1)
    def _():
        _deg_one_conv(s, d2, out2, zcol_h, idx_v, ones_v, sh)
        _deg_one_conv(s, d3, out3, zcol_h, idx_v, ones_v, sh)


_CPT_AGG = NCHUNK // NS


_NB = 2


def _agg_one_conv(s, g_ref, src_ref, dst_ref, out_ref,
                  sidx_v, didx_v, rows_v, acc_sh, gsems, ssems):
    pltpu.sync_copy(
        g_ref.at[pl.ds(s * ROWS_PER_TILE, ROWS_PER_TILE)],
        acc_sh.at[pl.ds(s * ROWS_PER_TILE, ROWS_PER_TILE)])
    plsc.subcore_barrier()

    pltpu.sync_copy(src_ref.at[pl.ds(s * _CPT_AGG, _CPT_AGG)], sidx_v)
    pltpu.sync_copy(dst_ref.at[pl.ds(s * _CPT_AGG, _CPT_AGG)], didx_v)

    for b in range(_NB):
        pltpu.async_copy(g_ref.at[sidx_v.at[b]], rows_v.at[b], gsems[b])

    @pl.loop(0, _CPT_AGG, step=_NB)
    def _(j):
        for b in range(_NB):
            jj = j + b
            pltpu.make_async_copy(
                g_ref.at[sidx_v.at[jj]], rows_v.at[b], gsems[b]).wait()
            pltpu.sync_copy(rows_v.at[b], acc_sh.at[didx_v.at[jj]], add=True)

            @pl.when(jj + _NB < _CPT_AGG)
            def _():
                pltpu.async_copy(
                    g_ref.at[sidx_v.at[jj + _NB]], rows_v.at[b], gsems[b])

    plsc.subcore_barrier()
    pltpu.sync_copy(
        acc_sh.at[pl.ds(s * ROWS_PER_TILE, ROWS_PER_TILE)],
        out_ref.at[pl.ds(s * ROWS_PER_TILE, ROWS_PER_TILE)],
    )
    plsc.subcore_barrier()


@functools.partial(
    pl.kernel,
    out_type=tuple(jax.ShapeDtypeStruct((N_PAD, D), jnp.float32) for _ in range(4)),
    mesh=_mesh,
    scratch_types=[
        pltpu.VMEM((_CPT_AGG, CHUNK), jnp.int32),
        pltpu.VMEM((_CPT_AGG, CHUNK), jnp.int32),
        pltpu.VMEM((_NB, CHUNK, D), jnp.float32),
        pltpu.VMEM_SHARED((N_PAD, D), jnp.float32),
    ] + [pltpu.SemaphoreType.DMA] * (2 * _NB),
)
def _agg_kernel(g0, g1, g2, g3, s0, s1, s2, s3, t0, t1, t2, t3,
                o0, o1, o2, o3,
                sidx_v, didx_v, rows_v, acc_sh, *sems):
    c = lax.axis_index("c")
    s = lax.axis_index("s")
    gsems, ssems = sems[:_NB], sems[_NB:]

    @pl.when(c == 0)
    def _():
        _agg_one_conv(s, g0, s0, t0, o0, sidx_v, didx_v, rows_v, acc_sh, gsems, ssems)
        _agg_one_conv(s, g1, s1, t1, o1, sidx_v, didx_v, rows_v, acc_sh, gsems, ssems)

    @pl.when(c == 1)
    def _():
        _agg_one_conv(s, g2, s2, t2, o2, sidx_v, didx_v, rows_v, acc_sh, gsems, ssems)
        _agg_one_conv(s, g3, s3, t3, o3, sidx_v, didx_v, rows_v, acc_sh, gsems, ssems)


BLK = 512


def _scale_body(x_ref, w1_ref, w2_ref, degp_ref, g0, g1, g2, g3):
    h1 = jnp.dot(x_ref[...], w1_ref[...], preferred_element_type=jnp.float32)
    h2 = jnp.dot(x_ref[...], w2_ref[...], preferred_element_type=jnp.float32)
    dinv = lax.rsqrt(degp_ref[...] + 1.0)
    g0[...] = h1 * dinv[0][:, None]
    g1[...] = h1 * dinv[1][:, None]
    g2[...] = h2 * dinv[2][:, None]
    g3[...] = h2 * dinv[3][:, None]


def _scale_call(x, W1, W2, degp):
    grid = (N_PAD // BLK,)
    gspec = pl.BlockSpec((BLK, D), lambda i: (i, 0))
    return pl.pallas_call(
        _scale_body,
        grid=grid,
        in_specs=[
            pl.BlockSpec((BLK, D), lambda i: (i, 0)),
            pl.BlockSpec((D, D), lambda i: (0, 0)),
            pl.BlockSpec((D, D), lambda i: (0, 0)),
            pl.BlockSpec((8, BLK), lambda i: (0, i)),
        ],
        out_specs=[gspec, gspec, gspec, gspec],
        out_shape=[jax.ShapeDtypeStruct((N_PAD, D), jnp.float32)] * 4,
    )(x, W1, W2, degp)


def _finish_body(a0, a1, a2, a3, degp_ref,
                 b1_ref, b2_ref, wt_top_ref, wt_bot_ref, bt_ref,
                 xa_ref, xb_ref):
    dinv = lax.rsqrt(degp_ref[...] + 1.0)
    relu = lambda v: jnp.maximum(v, 0.0)
    o0 = relu(dinv[0][:, None] * a0[...] + b1_ref[...])
    o1 = relu(dinv[1][:, None] * a1[...] + b1_ref[...])
    o2 = relu(dinv[2][:, None] * a2[...] + b2_ref[...])
    o3 = relu(dinv[3][:, None] * a3[...] + b2_ref[...])
    wt_top = wt_top_ref[...]
    wt_bot = wt_bot_ref[...]
    xa_ref[...] = relu(
        jnp.dot(o0, wt_top, preferred_element_type=jnp.float32)
        + jnp.dot(o2, wt_bot, preferred_element_type=jnp.float32)
        + bt_ref[...])
    xb_ref[...] = relu(
        jnp.dot(o1, wt_top, preferred_element_type=jnp.float32)
        + jnp.dot(o3, wt_bot, preferred_element_type=jnp.float32)
        + bt_ref[...])


def _finish_call(accs, degp, b1, b2, Wt, bt):
    grid = (N_PAD // BLK,)
    nspec = pl.BlockSpec((BLK, D), lambda i: (i, 0))
    wspec = pl.BlockSpec((D, D), lambda i: (0, 0))
    bspec = pl.BlockSpec((1, D), lambda i: (0, 0))
    return pl.pallas_call(
        _finish_body,
        grid=grid,
        in_specs=[nspec] * 4 + [
            pl.BlockSpec((8, BLK), lambda i: (0, i)),
            bspec, bspec, wspec, wspec, bspec,
        ],
        out_specs=[nspec, nspec],
        out_shape=[jax.ShapeDtypeStruct((N_PAD, D), jnp.float32)] * 2,
    )(*accs, degp, b1.reshape(1, D), b2.reshape(1, D),
      Wt[:D], Wt[D:], bt.reshape(1, D))


def _pad_edges(ei):
    src = ei[0].astype(jnp.int32).reshape(NCHUNK, CHUNK)
    dst = ei[1].astype(jnp.int32).reshape(NCHUNK, CHUNK)
    return src, dst


def kernel(x, view_a_pos, view_a_neg, view_b_pos, view_b_neg,
           W1, b1, W2, b2, Wt, bt):
    edges = [_pad_edges(v) for v in
             (view_a_pos, view_b_pos, view_a_neg, view_b_neg)]
    srcs = [e[0] for e in edges]
    dsts = [e[1] for e in edges]

    ones_h = jnp.ones((CHUNK,), jnp.float32)
    zcol_h = jnp.zeros((ROWS_PER_TILE,), jnp.float32)

    deg4 = _deg_kernel(*dsts, ones_h, zcol_h)
    degp = jnp.concatenate(
        [jnp.stack(deg4), jnp.zeros((4, N_PAD), jnp.float32)], axis=0)
    gs = _scale_call(x, W1, W2, degp)
    accs = _agg_kernel(*gs, *srcs, *dsts)
    xa, xb = _finish_call(accs, degp, b1, b2, Wt, bt)
    return xa[:N], xb[:N]


# speedup vs baseline: 2.0644x; 1.0788x over previous
import functools

import jax
import jax.numpy as jnp
from jax import lax
from jax.experimental import pallas as pl
from jax.experimental.pallas import tpu as pltpu
from jax.experimental.pallas import tpu_sc as plsc

N = 10000
D = 128
E = 80000

NC = 2
NS = 16
CHUNK = 125
N_PAD = 10240
NCHUNK = E // CHUNK
ROWS_PER_TILE = N_PAD // NS

_mesh = plsc.VectorSubcoreMesh(core_axis_name="c", subcore_axis_name="s")


_CPT_DEG = NCHUNK // NS


def _deg_one_conv(s, dref, out_ref, zcol_h, idx_v, ones_v, sh):
    pltpu.sync_copy(zcol_h, sh.at[pl.ds(s * ROWS_PER_TILE, ROWS_PER_TILE)])
    plsc.subcore_barrier()

    pltpu.sync_copy(dref.at[pl.ds(s * _CPT_DEG, _CPT_DEG)], idx_v)

    @pl.loop(0, _CPT_DEG)
    def _(j):
        pltpu.sync_copy(ones_v, sh.at[idx_v.at[j]], add=True)

    plsc.subcore_barrier()
    pltpu.sync_copy(
        sh.at[pl.ds(s * ROWS_PER_TILE, ROWS_PER_TILE)],
        out_ref.at[pl.ds(s * ROWS_PER_TILE, ROWS_PER_TILE)],
    )
    plsc.subcore_barrier()


@functools.partial(
    pl.kernel,
    out_type=tuple(jax.ShapeDtypeStruct((N_PAD,), jnp.float32) for _ in range(4)),
    mesh=_mesh,
    scratch_types=[
        pltpu.VMEM((_CPT_DEG, CHUNK), jnp.int32),
        pltpu.VMEM((CHUNK,), jnp.float32),
        pltpu.VMEM_SHARED((N_PAD,), jnp.float32),
    ],
)
def _deg_kernel(d0, d1, d2, d3, ones_h, zcol_h,
                out0, out1, out2, out3,
                idx_v, ones_v, sh):
    c = lax.axis_index("c")
    s = lax.axis_index("s")

    pltpu.sync_copy(ones_h, ones_v)

    @pl.when(c == 0)
    def _():
        _deg_one_conv(s, d0, out0, zcol_h, idx_v, ones_v, sh)
        _deg_one_conv(s, d1, out1, zcol_h, idx_v, ones_v, sh)

    @pl.when(c == 1)
    def _():
        _deg_one_conv(s, d2, out2, zcol_h, idx_v, ones_v, sh)
        _deg_one_conv(s, d3, out3, zcol_h, idx_v, ones_v, sh)


_CPT_AGG = NCHUNK // NS


_NB = 2


def _agg_one_conv(s, g_ref, src_ref, dst_ref, out_ref,
                  sidx_v, didx_v, rows_v, acc_sh, gsems, ssems):
    pltpu.sync_copy(
        g_ref.at[pl.ds(s * ROWS_PER_TILE, ROWS_PER_TILE)],
        acc_sh.at[pl.ds(s * ROWS_PER_TILE, ROWS_PER_TILE)])
    plsc.subcore_barrier()

    pltpu.sync_copy(src_ref.at[pl.ds(s * _CPT_AGG, _CPT_AGG)], sidx_v)
    pltpu.sync_copy(dst_ref.at[pl.ds(s * _CPT_AGG, _CPT_AGG)], didx_v)

    for b in range(_NB):
        pltpu.async_copy(g_ref.at[sidx_v.at[b]], rows_v.at[b], gsems[b])

    @pl.loop(0, _CPT_AGG, step=_NB)
    def _(j):
        for b in range(_NB):
            jj = j + b
            pltpu.make_async_copy(
                g_ref.at[sidx_v.at[jj]], rows_v.at[b], gsems[b]).wait()
            pltpu.sync_copy(rows_v.at[b], acc_sh.at[didx_v.at[jj]], add=True)

            @pl.when(jj + _NB < _CPT_AGG)
            def _():
                pltpu.async_copy(
                    g_ref.at[sidx_v.at[jj + _NB]], rows_v.at[b], gsems[b])

    plsc.subcore_barrier()
    pltpu.sync_copy(
        acc_sh.at[pl.ds(s * ROWS_PER_TILE, ROWS_PER_TILE)],
        out_ref.at[pl.ds(s * ROWS_PER_TILE, ROWS_PER_TILE)],
    )
    plsc.subcore_barrier()


@functools.partial(
    pl.kernel,
    out_type=tuple(jax.ShapeDtypeStruct((N_PAD, D), jnp.float32) for _ in range(4)),
    mesh=_mesh,
    scratch_types=[
        pltpu.VMEM((_CPT_AGG, CHUNK), jnp.int32),
        pltpu.VMEM((_CPT_AGG, CHUNK), jnp.int32),
        pltpu.VMEM((_NB, CHUNK, D), jnp.float32),
        pltpu.VMEM_SHARED((N_PAD, D), jnp.float32),
    ] + [pltpu.SemaphoreType.DMA] * (2 * _NB),
)
def _agg_kernel(g0, g1, g2, g3, s0, s1, s2, s3, t0, t1, t2, t3,
                o0, o1, o2, o3,
                sidx_v, didx_v, rows_v, acc_sh, *sems):
    c = lax.axis_index("c")
    s = lax.axis_index("s")
    gsems, ssems = sems[:_NB], sems[_NB:]

    @pl.when(c == 0)
    def _():
        _agg_one_conv(s, g0, s0, t0, o0, sidx_v, didx_v, rows_v, acc_sh, gsems, ssems)
        _agg_one_conv(s, g1, s1, t1, o1, sidx_v, didx_v, rows_v, acc_sh, gsems, ssems)

    @pl.when(c == 1)
    def _():
        _agg_one_conv(s, g2, s2, t2, o2, sidx_v, didx_v, rows_v, acc_sh, gsems, ssems)
        _agg_one_conv(s, g3, s3, t3, o3, sidx_v, didx_v, rows_v, acc_sh, gsems, ssems)


BLK = 2048


def _scale_body(x_ref, w1_ref, w2_ref, degp_ref, g0, g1, g2, g3):
    h1 = jnp.dot(x_ref[...], w1_ref[...], preferred_element_type=jnp.float32)
    h2 = jnp.dot(x_ref[...], w2_ref[...], preferred_element_type=jnp.float32)
    dinv = lax.rsqrt(degp_ref[...] + 1.0)
    g0[...] = h1 * dinv[0][:, None]
    g1[...] = h1 * dinv[1][:, None]
    g2[...] = h2 * dinv[2][:, None]
    g3[...] = h2 * dinv[3][:, None]


def _scale_call(x, W1, W2, degp):
    grid = (N_PAD // BLK,)
    gspec = pl.BlockSpec((BLK, D), lambda i: (i, 0))
    return pl.pallas_call(
        _scale_body,
        grid=grid,
        in_specs=[
            pl.BlockSpec((BLK, D), lambda i: (i, 0)),
            pl.BlockSpec((D, D), lambda i: (0, 0)),
            pl.BlockSpec((D, D), lambda i: (0, 0)),
            pl.BlockSpec((8, BLK), lambda i: (0, i)),
        ],
        out_specs=[gspec, gspec, gspec, gspec],
        out_shape=[jax.ShapeDtypeStruct((N_PAD, D), jnp.float32)] * 4,
    )(x, W1, W2, degp)


def _finish_body(a0, a1, a2, a3, degp_ref,
                 b1_ref, b2_ref, wt_top_ref, wt_bot_ref, bt_ref,
                 xa_ref, xb_ref):
    dinv = lax.rsqrt(degp_ref[...] + 1.0)
    relu = lambda v: jnp.maximum(v, 0.0)
    o0 = relu(dinv[0][:, None] * a0[...] + b1_ref[...])
    o1 = relu(dinv[1][:, None] * a1[...] + b1_ref[...])
    o2 = relu(dinv[2][:, None] * a2[...] + b2_ref[...])
    o3 = relu(dinv[3][:, None] * a3[...] + b2_ref[...])
    wt_top = wt_top_ref[...]
    wt_bot = wt_bot_ref[...]
    xa_ref[...] = relu(
        jnp.dot(o0, wt_top, preferred_element_type=jnp.float32)
        + jnp.dot(o2, wt_bot, preferred_element_type=jnp.float32)
        + bt_ref[...])
    xb_ref[...] = relu(
        jnp.dot(o1, wt_top, preferred_element_type=jnp.float32)
        + jnp.dot(o3, wt_bot, preferred_element_type=jnp.float32)
        + bt_ref[...])


def _finish_call(accs, degp, b1, b2, Wt, bt):
    grid = (N_PAD // BLK,)
    nspec = pl.BlockSpec((BLK, D), lambda i: (i, 0))
    wspec = pl.BlockSpec((D, D), lambda i: (0, 0))
    bspec = pl.BlockSpec((1, D), lambda i: (0, 0))
    return pl.pallas_call(
        _finish_body,
        grid=grid,
        in_specs=[nspec] * 4 + [
            pl.BlockSpec((8, BLK), lambda i: (0, i)),
            bspec, bspec, wspec, wspec, bspec,
        ],
        out_specs=[nspec, nspec],
        out_shape=[jax.ShapeDtypeStruct((N_PAD, D), jnp.float32)] * 2,
    )(*accs, degp, b1.reshape(1, D), b2.reshape(1, D),
      Wt[:D], Wt[D:], bt.reshape(1, D))


def _pad_edges(ei):
    src = ei[0].astype(jnp.int32).reshape(NCHUNK, CHUNK)
    dst = ei[1].astype(jnp.int32).reshape(NCHUNK, CHUNK)
    return src, dst


def kernel(x, view_a_pos, view_a_neg, view_b_pos, view_b_neg,
           W1, b1, W2, b2, Wt, bt):
    edges = [_pad_edges(v) for v in
             (view_a_pos, view_b_pos, view_a_neg, view_b_neg)]
    srcs = [e[0] for e in edges]
    dsts = [e[1] for e in edges]

    ones_h = jnp.ones((CHUNK,), jnp.float32)
    zcol_h = jnp.zeros((ROWS_PER_TILE,), jnp.float32)

    deg4 = _deg_kernel(*dsts, ones_h, zcol_h)
    degp = jnp.concatenate(
        [jnp.stack(deg4), jnp.zeros((4, N_PAD), jnp.float32)], axis=0)
    gs = _scale_call(x, W1, W2, degp)
    accs = _agg_kernel(*gs, *srcs, *dsts)
    xa, xb = _finish_call(accs, degp, b1, b2, Wt, bt)
    return xa[:N], xb[:N]


# speedup vs baseline: 2.0758x; 1.0055x over previous
import functools

import jax
import jax.numpy as jnp
from jax import lax
from jax.experimental import pallas as pl
from jax.experimental.pallas import tpu as pltpu
from jax.experimental.pallas import tpu_sc as plsc

N = 10000
D = 128
E = 80000

NC = 2
NS = 16
CHUNK = 125
N_PAD = 10240
NCHUNK = E // CHUNK
ROWS_PER_TILE = N_PAD // NS

_mesh = plsc.VectorSubcoreMesh(core_axis_name="c", subcore_axis_name="s")


_CPT_DEG = NCHUNK // NS


def _deg_one_conv(s, dref, out_ref, zcol_h, idx_v, ones_v, sh):
    pltpu.sync_copy(zcol_h, sh.at[pl.ds(s * ROWS_PER_TILE, ROWS_PER_TILE)])
    plsc.subcore_barrier()

    pltpu.sync_copy(dref.at[pl.ds(s * _CPT_DEG, _CPT_DEG)], idx_v)

    @pl.loop(0, _CPT_DEG)
    def _(j):
        pltpu.sync_copy(ones_v, sh.at[idx_v.at[j]], add=True)

    plsc.subcore_barrier()
    pltpu.sync_copy(
        sh.at[pl.ds(s * ROWS_PER_TILE, ROWS_PER_TILE)],
        out_ref.at[pl.ds(s * ROWS_PER_TILE, ROWS_PER_TILE)],
    )
    plsc.subcore_barrier()


@functools.partial(
    pl.kernel,
    out_type=tuple(jax.ShapeDtypeStruct((N_PAD,), jnp.float32) for _ in range(4)),
    mesh=_mesh,
    scratch_types=[
        pltpu.VMEM((_CPT_DEG, CHUNK), jnp.int32),
        pltpu.VMEM((CHUNK,), jnp.float32),
        pltpu.VMEM_SHARED((N_PAD,), jnp.float32),
    ],
)
def _deg_kernel(d0, d1, d2, d3, ones_h, zcol_h,
                out0, out1, out2, out3,
                idx_v, ones_v, sh):
    c = lax.axis_index("c")
    s = lax.axis_index("s")

    pltpu.sync_copy(ones_h, ones_v)

    @pl.when(c == 0)
    def _():
        _deg_one_conv(s, d0, out0, zcol_h, idx_v, ones_v, sh)
        _deg_one_conv(s, d1, out1, zcol_h, idx_v, ones_v, sh)

    @pl.when(c == 1)
    def _():
        _deg_one_conv(s, d2, out2, zcol_h, idx_v, ones_v, sh)
        _deg_one_conv(s, d3, out3, zcol_h, idx_v, ones_v, sh)


_CPT_AGG = NCHUNK // NS


_NB = 2


def _agg_one_conv(s, g_ref, src_ref, dst_ref, out_ref,
                  sidx_v, didx_v, rows_v, acc_sh, gsems, ssems):
    pltpu.sync_copy(
        g_ref.at[pl.ds(s * ROWS_PER_TILE, ROWS_PER_TILE)],
        acc_sh.at[pl.ds(s * ROWS_PER_TILE, ROWS_PER_TILE)])
    plsc.subcore_barrier()

    pltpu.sync_copy(src_ref.at[pl.ds(s * _CPT_AGG, _CPT_AGG)], sidx_v)
    pltpu.sync_copy(dst_ref.at[pl.ds(s * _CPT_AGG, _CPT_AGG)], didx_v)

    for b in range(_NB):
        pltpu.async_copy(g_ref.at[sidx_v.at[b]], rows_v.at[b], gsems[b])

    @pl.loop(0, _CPT_AGG, step=_NB)
    def _(j):
        for b in range(_NB):
            jj = j + b
            pltpu.make_async_copy(
                g_ref.at[sidx_v.at[jj]], rows_v.at[b], gsems[b]).wait()
            pltpu.sync_copy(rows_v.at[b], acc_sh.at[didx_v.at[jj]], add=True)

            @pl.when(jj + _NB < _CPT_AGG)
            def _():
                pltpu.async_copy(
                    g_ref.at[sidx_v.at[jj + _NB]], rows_v.at[b], gsems[b])

    plsc.subcore_barrier()
    pltpu.sync_copy(
        acc_sh.at[pl.ds(s * ROWS_PER_TILE, ROWS_PER_TILE)],
        out_ref.at[pl.ds(s * ROWS_PER_TILE, ROWS_PER_TILE)],
    )
    plsc.subcore_barrier()


@functools.partial(
    pl.kernel,
    out_type=tuple(jax.ShapeDtypeStruct((N_PAD, D), jnp.float32) for _ in range(4)),
    mesh=_mesh,
    scratch_types=[
        pltpu.VMEM((_CPT_AGG, CHUNK), jnp.int32),
        pltpu.VMEM((_CPT_AGG, CHUNK), jnp.int32),
        pltpu.VMEM((_NB, CHUNK, D), jnp.float32),
        pltpu.VMEM_SHARED((N_PAD, D), jnp.float32),
    ] + [pltpu.SemaphoreType.DMA] * (2 * _NB),
)
def _agg_kernel(g0, g1, g2, g3, s0, s1, s2, s3, t0, t1, t2, t3,
                o0, o1, o2, o3,
                sidx_v, didx_v, rows_v, acc_sh, *sems):
    c = lax.axis_index("c")
    s = lax.axis_index("s")
    gsems, ssems = sems[:_NB], sems[_NB:]

    @pl.when(c == 0)
    def _():
        _agg_one_conv(s, g0, s0, t0, o0, sidx_v, didx_v, rows_v, acc_sh, gsems, ssems)
        _agg_one_conv(s, g1, s1, t1, o1, sidx_v, didx_v, rows_v, acc_sh, gsems, ssems)

    @pl.when(c == 1)
    def _():
        _agg_one_conv(s, g2, s2, t2, o2, sidx_v, didx_v, rows_v, acc_sh, gsems, ssems)
        _agg_one_conv(s, g3, s3, t3, o3, sidx_v, didx_v, rows_v, acc_sh, gsems, ssems)


BLK = 5120


def _scale_body(x_ref, w1_ref, w2_ref, degp_ref, g0, g1, g2, g3):
    h1 = jnp.dot(x_ref[...], w1_ref[...], preferred_element_type=jnp.float32)
    h2 = jnp.dot(x_ref[...], w2_ref[...], preferred_element_type=jnp.float32)
    dinv = lax.rsqrt(degp_ref[...] + 1.0)
    g0[...] = h1 * dinv[0][:, None]
    g1[...] = h1 * dinv[1][:, None]
    g2[...] = h2 * dinv[2][:, None]
    g3[...] = h2 * dinv[3][:, None]


def _scale_call(x, W1, W2, degp):
    grid = (N_PAD // BLK,)
    gspec = pl.BlockSpec((BLK, D), lambda i: (i, 0))
    return pl.pallas_call(
        _scale_body,
        grid=grid,
        in_specs=[
            pl.BlockSpec((BLK, D), lambda i: (i, 0)),
            pl.BlockSpec((D, D), lambda i: (0, 0)),
            pl.BlockSpec((D, D), lambda i: (0, 0)),
            pl.BlockSpec((8, BLK), lambda i: (0, i)),
        ],
        out_specs=[gspec, gspec, gspec, gspec],
        out_shape=[jax.ShapeDtypeStruct((N_PAD, D), jnp.float32)] * 4,
    )(x, W1, W2, degp)


def _finish_body(a0, a1, a2, a3, degp_ref,
                 b1_ref, b2_ref, wt_top_ref, wt_bot_ref, bt_ref,
                 xa_ref, xb_ref):
    dinv = lax.rsqrt(degp_ref[...] + 1.0)
    relu = lambda v: jnp.maximum(v, 0.0)
    o0 = relu(dinv[0][:, None] * a0[...] + b1_ref[...])
    o1 = relu(dinv[1][:, None] * a1[...] + b1_ref[...])
    o2 = relu(dinv[2][:, None] * a2[...] + b2_ref[...])
    o3 = relu(dinv[3][:, None] * a3[...] + b2_ref[...])
    wt_top = wt_top_ref[...]
    wt_bot = wt_bot_ref[...]
    xa_ref[...] = relu(
        jnp.dot(o0, wt_top, preferred_element_type=jnp.float32)
        + jnp.dot(o2, wt_bot, preferred_element_type=jnp.float32)
        + bt_ref[...])
    xb_ref[...] = relu(
        jnp.dot(o1, wt_top, preferred_element_type=jnp.float32)
        + jnp.dot(o3, wt_bot, preferred_element_type=jnp.float32)
        + bt_ref[...])


def _finish_call(accs, degp, b1, b2, Wt, bt):
    grid = (N_PAD // BLK,)
    nspec = pl.BlockSpec((BLK, D), lambda i: (i, 0))
    wspec = pl.BlockSpec((D, D), lambda i: (0, 0))
    bspec = pl.BlockSpec((1, D), lambda i: (0, 0))
    return pl.pallas_call(
        _finish_body,
        grid=grid,
        in_specs=[nspec] * 4 + [
            pl.BlockSpec((8, BLK), lambda i: (0, i)),
            bspec, bspec, wspec, wspec, bspec,
        ],
        out_specs=[nspec, nspec],
        out_shape=[jax.ShapeDtypeStruct((N_PAD, D), jnp.float32)] * 2,
    )(*accs, degp, b1.reshape(1, D), b2.reshape(1, D),
      Wt[:D], Wt[D:], bt.reshape(1, D))


def _pad_edges(ei):
    src = ei[0].astype(jnp.int32).reshape(NCHUNK, CHUNK)
    dst = ei[1].astype(jnp.int32).reshape(NCHUNK, CHUNK)
    return src, dst


def kernel(x, view_a_pos, view_a_neg, view_b_pos, view_b_neg,
           W1, b1, W2, b2, Wt, bt):
    edges = [_pad_edges(v) for v in
             (view_a_pos, view_b_pos, view_a_neg, view_b_neg)]
    srcs = [e[0] for e in edges]
    dsts = [e[1] for e in edges]

    ones_h = jnp.ones((CHUNK,), jnp.float32)
    zcol_h = jnp.zeros((ROWS_PER_TILE,), jnp.float32)

    deg4 = _deg_kernel(*dsts, ones_h, zcol_h)
    degp = jnp.concatenate(
        [jnp.stack(deg4), jnp.zeros((4, N_PAD), jnp.float32)], axis=0)
    gs = _scale_call(x, W1, W2, degp)
    accs = _agg_kernel(*gs, *srcs, *dsts)
    xa, xb = _finish_call(accs, degp, b1, b2, Wt, bt)
    return xa[:N], xb[:N]


# speedup vs baseline: 2.1556x; 1.0384x over previous
import functools

import jax
import jax.numpy as jnp
from jax import lax
from jax.experimental import pallas as pl
from jax.experimental.pallas import tpu as pltpu
from jax.experimental.pallas import tpu_sc as plsc

N = 10000
D = 128
E = 80000

NC = 2
NS = 16
CHUNK = 125
N_PAD = 10240
NCHUNK = E // CHUNK
ROWS_PER_TILE = N_PAD // NS

_mesh = plsc.VectorSubcoreMesh(core_axis_name="c", subcore_axis_name="s")


_CPT_DEG = NCHUNK // NS


def _deg_one_conv(s, dref, out_ref, zcol_h, idx_v, ones_v, sh):
    pltpu.sync_copy(zcol_h, sh.at[pl.ds(s * ROWS_PER_TILE, ROWS_PER_TILE)])
    plsc.subcore_barrier()

    pltpu.sync_copy(dref.at[pl.ds(s * _CPT_DEG, _CPT_DEG)], idx_v)

    @pl.loop(0, _CPT_DEG)
    def _(j):
        pltpu.sync_copy(ones_v, sh.at[idx_v.at[j]], add=True)

    plsc.subcore_barrier()
    pltpu.sync_copy(
        sh.at[pl.ds(s * ROWS_PER_TILE, ROWS_PER_TILE)],
        out_ref.at[pl.ds(s * ROWS_PER_TILE, ROWS_PER_TILE)],
    )
    plsc.subcore_barrier()


@functools.partial(
    pl.kernel,
    out_type=tuple(jax.ShapeDtypeStruct((N_PAD,), jnp.float32) for _ in range(4)),
    mesh=_mesh,
    scratch_types=[
        pltpu.VMEM((_CPT_DEG, CHUNK), jnp.int32),
        pltpu.VMEM((CHUNK,), jnp.float32),
        pltpu.VMEM_SHARED((N_PAD,), jnp.float32),
    ],
)
def _deg_kernel(d0, d1, d2, d3, ones_h, zcol_h,
                out0, out1, out2, out3,
                idx_v, ones_v, sh):
    c = lax.axis_index("c")
    s = lax.axis_index("s")

    pltpu.sync_copy(ones_h, ones_v)

    @pl.when(c == 0)
    def _():
        _deg_one_conv(s, d0, out0, zcol_h, idx_v, ones_v, sh)
        _deg_one_conv(s, d1, out1, zcol_h, idx_v, ones_v, sh)

    @pl.when(c == 1)
    def _():
        _deg_one_conv(s, d2, out2, zcol_h, idx_v, ones_v, sh)
        _deg_one_conv(s, d3, out3, zcol_h, idx_v, ones_v, sh)


_CPT_AGG = NCHUNK // NS


_NB = 2


def _agg_one_conv(s, g_ref, src_ref, dst_ref, out_ref,
                  sidx_v, didx_v, rows_v, acc_sh, gsems, ssems):
    pltpu.sync_copy(
        g_ref.at[pl.ds(s * ROWS_PER_TILE, ROWS_PER_TILE)],
        acc_sh.at[pl.ds(s * ROWS_PER_TILE, ROWS_PER_TILE)])
    plsc.subcore_barrier()

    pltpu.sync_copy(src_ref.at[pl.ds(s * _CPT_AGG, _CPT_AGG)], sidx_v)
    pltpu.sync_copy(dst_ref.at[pl.ds(s * _CPT_AGG, _CPT_AGG)], didx_v)

    for b in range(_NB):
        pltpu.async_copy(g_ref.at[sidx_v.at[b]], rows_v.at[b], gsems[b])

    @pl.loop(0, _CPT_AGG, step=_NB)
    def _(j):
        for b in range(_NB):
            jj = j + b
            pltpu.make_async_copy(
                g_ref.at[sidx_v.at[jj]], rows_v.at[b], gsems[b]).wait()
            pltpu.sync_copy(rows_v.at[b], acc_sh.at[didx_v.at[jj]], add=True)

            @pl.when(jj + _NB < _CPT_AGG)
            def _():
                pltpu.async_copy(
                    g_ref.at[sidx_v.at[jj + _NB]], rows_v.at[b], gsems[b])

    plsc.subcore_barrier()
    pltpu.sync_copy(
        acc_sh.at[pl.ds(s * ROWS_PER_TILE, ROWS_PER_TILE)],
        out_ref.at[pl.ds(s * ROWS_PER_TILE, ROWS_PER_TILE)],
    )
    plsc.subcore_barrier()


@functools.partial(
    pl.kernel,
    out_type=tuple(jax.ShapeDtypeStruct((N_PAD, D), jnp.float32) for _ in range(4)),
    mesh=_mesh,
    scratch_types=[
        pltpu.VMEM((_CPT_AGG, CHUNK), jnp.int32),
        pltpu.VMEM((_CPT_AGG, CHUNK), jnp.int32),
        pltpu.VMEM((_NB, CHUNK, D), jnp.float32),
        pltpu.VMEM_SHARED((N_PAD, D), jnp.float32),
    ] + [pltpu.SemaphoreType.DMA] * (2 * _NB),
)
def _agg_kernel(g0, g1, g2, g3, s0, s1, s2, s3, t0, t1, t2, t3,
                o0, o1, o2, o3,
                sidx_v, didx_v, rows_v, acc_sh, *sems):
    c = lax.axis_index("c")
    s = lax.axis_index("s")
    gsems, ssems = sems[:_NB], sems[_NB:]

    @pl.when(c == 0)
    def _():
        _agg_one_conv(s, g0, s0, t0, o0, sidx_v, didx_v, rows_v, acc_sh, gsems, ssems)
        _agg_one_conv(s, g1, s1, t1, o1, sidx_v, didx_v, rows_v, acc_sh, gsems, ssems)

    @pl.when(c == 1)
    def _():
        _agg_one_conv(s, g2, s2, t2, o2, sidx_v, didx_v, rows_v, acc_sh, gsems, ssems)
        _agg_one_conv(s, g3, s3, t3, o3, sidx_v, didx_v, rows_v, acc_sh, gsems, ssems)


BLK = 5120


def _scale_body(x_ref, w1_ref, w2_ref, degp_ref, g0, g1, g2, g3):
    h1 = jnp.dot(x_ref[...], w1_ref[...], preferred_element_type=jnp.float32)
    h2 = jnp.dot(x_ref[...], w2_ref[...], preferred_element_type=jnp.float32)
    dinv = lax.rsqrt(degp_ref[...] + 1.0)
    g0[...] = h1 * dinv[0][:, None]
    g1[...] = h1 * dinv[1][:, None]
    g2[...] = h2 * dinv[2][:, None]
    g3[...] = h2 * dinv[3][:, None]


def _scale_call(x, W1, W2, degp):
    grid = (N_PAD // BLK,)
    gspec = pl.BlockSpec((BLK, D), lambda i: (i, 0))
    return pl.pallas_call(
        _scale_body,
        grid=grid,
        in_specs=[
            pl.BlockSpec((BLK, D), lambda i: (i, 0)),
            pl.BlockSpec((D, D), lambda i: (0, 0)),
            pl.BlockSpec((D, D), lambda i: (0, 0)),
            pl.BlockSpec((8, BLK), lambda i: (0, i)),
        ],
        out_specs=[gspec, gspec, gspec, gspec],
        out_shape=[jax.ShapeDtypeStruct((N_PAD, D), jnp.float32)] * 4,
    )(x, W1, W2, degp)


def _finish_body(a0, a1, a2, a3, degp_ref,
                 b1_ref, b2_ref, wt_top_ref, wt_bot_ref, bt_ref,
                 xa_ref, xb_ref):
    dinv = lax.rsqrt(degp_ref[...] + 1.0)
    relu = lambda v: jnp.maximum(v, 0.0)
    o0 = relu(dinv[0][:, None] * a0[...] + b1_ref[...])
    o1 = relu(dinv[1][:, None] * a1[...] + b1_ref[...])
    o2 = relu(dinv[2][:, None] * a2[...] + b2_ref[...])
    o3 = relu(dinv[3][:, None] * a3[...] + b2_ref[...])
    wt_top = wt_top_ref[...]
    wt_bot = wt_bot_ref[...]
    xa_ref[...] = relu(
        jnp.dot(o0, wt_top, preferred_element_type=jnp.float32)
        + jnp.dot(o2, wt_bot, preferred_element_type=jnp.float32)
        + bt_ref[...])
    xb_ref[...] = relu(
        jnp.dot(o1, wt_top, preferred_element_type=jnp.float32)
        + jnp.dot(o3, wt_bot, preferred_element_type=jnp.float32)
        + bt_ref[...])


def _finish_call(accs, degp, b1, b2, Wt, bt):
    grid = (N_PAD // BLK,)
    nspec = pl.BlockSpec((BLK, D), lambda i: (i, 0))
    wspec = pl.BlockSpec((D, D), lambda i: (0, 0))
    bspec = pl.BlockSpec((1, D), lambda i: (0, 0))
    return pl.pallas_call(
        _finish_body,
        grid=grid,
        in_specs=[nspec] * 4 + [
            pl.BlockSpec((8, BLK), lambda i: (0, i)),
            bspec, bspec, wspec, wspec, bspec,
        ],
        out_specs=[nspec, nspec],
        out_shape=[jax.ShapeDtypeStruct((N, D), jnp.float32)] * 2,
    )(*accs, degp, b1.reshape(1, D), b2.reshape(1, D),
      Wt[:D], Wt[D:], bt.reshape(1, D))


def _pad_edges(ei):
    src = ei[0].astype(jnp.int32).reshape(NCHUNK, CHUNK)
    dst = ei[1].astype(jnp.int32).reshape(NCHUNK, CHUNK)
    return src, dst


def kernel(x, view_a_pos, view_a_neg, view_b_pos, view_b_neg,
           W1, b1, W2, b2, Wt, bt):
    edges = [_pad_edges(v) for v in
             (view_a_pos, view_b_pos, view_a_neg, view_b_neg)]
    srcs = [e[0] for e in edges]
    dsts = [e[1] for e in edges]

    ones_h = jnp.ones((CHUNK,), jnp.float32)
    zcol_h = jnp.zeros((ROWS_PER_TILE,), jnp.float32)

    deg4 = _deg_kernel(*dsts, ones_h, zcol_h)
    degp = jnp.concatenate(
        [jnp.stack(deg4), jnp.zeros((4, N_PAD), jnp.float32)], axis=0)
    gs = _scale_call(x, W1, W2, degp)
    accs = _agg_kernel(*gs, *srcs, *dsts)
    xa, xb = _finish_call(accs, degp, b1, b2, Wt, bt)
    return xa, xb
